# Initial kernel scaffold; baseline (speedup 1.0000x reference)
import jax
import jax.numpy as jnp
from jax.experimental import pallas as pl


def kernel(x, edge_attr, bus_shunt, W_enc_x, b_enc_x, W_enc_e, b_enc_e, W_edge, b_edge, W_node, b_node, W_dec, b_dec, edge_index, slack_idx):
    raise NotImplementedError("write your pallas kernel here")


# speedup vs baseline: 11.3650x; 11.3650x over previous
import jax
import jax.numpy as jnp
from jax import lax
from jax.experimental import pallas as pl
from jax.experimental.pallas import tpu as pltpu
from jax.experimental.pallas import tpu_sc as plsc

F32 = jnp.float32
NC, NS = 2, 16
NW = NC * NS
CH = 40
NBUF = 5
EB = 512
RPT = 320
SC_CH = 120


def _mesh():
    return plsc.VectorSubcoreMesh(core_axis_name="c", subcore_axis_name="s",
                                  num_cores=NC, num_subcores=NS)


def _gather_pair(tab, src3, dst3):
    D = tab.shape[1]
    _, n_chunks, _ = src3.shape
    epw = n_chunks * CH
    E = NW * epw
    n_groups = n_chunks // NBUF

    def body(t_hbm, s_hbm, d_hbm, os_hbm, od_hbm, ia, ib, *rest):
        bufa = rest[:NBUF]
        bufb = rest[NBUF:2 * NBUF]
        sema = rest[2 * NBUF:3 * NBUF]
        semb = rest[3 * NBUF:]
        w = lax.axis_index("s") * NC + lax.axis_index("c")
        pltpu.sync_copy(s_hbm.at[w], ia)
        pltpu.sync_copy(d_hbm.at[w], ib)
        base = w * epw

        def grp(g, carry):
            j0 = g * NBUF
            for b in range(NBUF):
                pltpu.async_copy(t_hbm.at[ia.at[j0 + b]], bufa[b], sema[b])
                pltpu.async_copy(t_hbm.at[ib.at[j0 + b]], bufb[b], semb[b])
            for b in range(NBUF):
                off = base + (j0 + b) * CH
                pltpu.make_async_copy(t_hbm.at[ia.at[j0 + b]], bufa[b],
                                      sema[b]).wait()
                pltpu.async_copy(bufa[b], os_hbm.at[pl.ds(off, CH)], sema[b])
                pltpu.make_async_copy(t_hbm.at[ib.at[j0 + b]], bufb[b],
                                      semb[b]).wait()
                pltpu.async_copy(bufb[b], od_hbm.at[pl.ds(off, CH)], semb[b])
            for b in range(NBUF):
                off = base + (j0 + b) * CH
                pltpu.make_async_copy(bufa[b], os_hbm.at[pl.ds(off, CH)],
                                      sema[b]).wait()
                pltpu.make_async_copy(bufb[b], od_hbm.at[pl.ds(off, CH)],
                                      semb[b]).wait()
            return carry

        lax.fori_loop(0, n_groups, grp, 0)

    f = pl.kernel(body,
                  out_type=[jax.ShapeDtypeStruct((E, D), F32)] * 2,
                  mesh=_mesh(),
                  scratch_types=([pltpu.VMEM((n_chunks, CH), jnp.int32)] * 2
                                 + [pltpu.VMEM((CH, D), F32)] * (2 * NBUF)
                                 + [pltpu.SemaphoreType.DMA] * (2 * NBUF)))
    return f(tab, src3, dst3)


def _scatter_sorted(vals, perm_pad, dsort_pad, sbounds):
    E, D = vals.shape
    NP2 = NW * RPT
    C8 = SC_CH + 8

    def body(v_hbm, p_hbm, d_hbm, sb_hbm, out_hbm,
             accf, buf, pidx, dsv, sbv, sem):
        w = lax.axis_index("s") * NC + lax.axis_index("c")
        pltpu.sync_copy(sb_hbm, sbv)
        r0 = w * RPT
        sbw = sbv[pl.ds(w, 16)]
        p0 = sbw[0]
        pend = sbw[1]

        def zstep(i, carry):
            accf[pl.ds(i * 16, 16)] = jnp.zeros((16,), F32)
            return carry

        lax.fori_loop(0, RPT * D // 16, zstep, 0)

        maxch = E // SC_CH + 2

        def chunk_body(ci, p):
            def active(pp):
                pa = (pp // 8) * 8
                lead = pp - pa
                pltpu.sync_copy(p_hbm.at[pl.ds(pa, C8)], pidx)
                pltpu.sync_copy(d_hbm.at[pl.ds(pa, C8)], dsv.at[pl.ds(0, C8)])
                pltpu.async_copy(v_hbm.at[pidx], buf, sem).wait()
                dn = jnp.minimum(SC_CH, pend - pp)

                def estep(i, carry):
                    eo = lead + i
                    rl = dsv[pl.ds(eo, 16)][0] - r0
                    rl = jnp.minimum(jnp.maximum(rl, 0), RPT - 1)
                    act = i < dn
                    bb = rl * D
                    for j in range(D // 16):
                        v = buf[eo, pl.ds(j * 16, 16)]
                        v = jnp.where(act, v, jnp.zeros((16,), F32))
                        plsc.addupdate(accf.at[pl.ds(bb + j * 16, 16)], v)
                    return carry

                lax.fori_loop(0, SC_CH, estep, 0)
                return pp + dn

            return lax.cond(p < pend, active, lambda q: q, p)

        lax.fori_loop(0, maxch, chunk_body, p0)
        pltpu.sync_copy(accf, out_hbm.at[w])

    f = pl.kernel(body,
                  out_type=jax.ShapeDtypeStruct((NW, RPT * D), F32),
                  mesh=_mesh(),
                  scratch_types=[pltpu.VMEM((RPT * D,), F32),
                                 pltpu.VMEM((C8, D), F32),
                                 pltpu.VMEM((C8,), jnp.int32),
                                 pltpu.VMEM((C8 + 16,), jnp.int32),
                                 pltpu.VMEM((NW + 16,), jnp.int32),
                                 pltpu.SemaphoreType.DMA])
    return f(vals, perm_pad, dsort_pad, sbounds)


def _gather_volt(vt, srcW, dstW):
    Nn = vt.shape[1]
    _, epw = srcW.shape
    E = NW * epw
    n16 = epw // 16

    def body(vt_hbm, s_hbm, d_hbm, ori_hbm, oii_hbm, orj_hbm, oij_hbm,
             vre_v, vim_v, isv, idv, bri, bii, brj, bij):
        w = lax.axis_index("s") * NC + lax.axis_index("c")
        pltpu.sync_copy(vt_hbm.at[0], vre_v)
        pltpu.sync_copy(vt_hbm.at[1], vim_v)
        pltpu.sync_copy(s_hbm.at[w], isv)
        pltpu.sync_copy(d_hbm.at[w], idv)

        def step(k, carry):
            sl = pl.ds(k * 16, 16)
            si = isv[sl]
            di = idv[sl]
            bri[sl] = plsc.load_gather(vre_v, [si])
            bii[sl] = plsc.load_gather(vim_v, [si])
            brj[sl] = plsc.load_gather(vre_v, [di])
            bij[sl] = plsc.load_gather(vim_v, [di])
            return carry

        lax.fori_loop(0, n16, step, 0)
        base = w * epw
        pltpu.sync_copy(bri, ori_hbm.at[pl.ds(base, epw)])
        pltpu.sync_copy(bii, oii_hbm.at[pl.ds(base, epw)])
        pltpu.sync_copy(brj, orj_hbm.at[pl.ds(base, epw)])
        pltpu.sync_copy(bij, oij_hbm.at[pl.ds(base, epw)])

    f = pl.kernel(body,
                  out_type=[jax.ShapeDtypeStruct((E,), F32)] * 4,
                  mesh=_mesh(),
                  compiler_params=pltpu.CompilerParams(
                      needs_layout_passes=False),
                  scratch_types=([pltpu.VMEM((Nn,), F32)] * 2
                                 + [pltpu.VMEM((epw,), jnp.int32)] * 2
                                 + [pltpu.VMEM((epw,), F32)] * 4))
    return f(vt, srcW, dstW)


def _dot(a, b):
    return jnp.dot(a, b, preferred_element_type=F32)


def _encode_nodes(x, W_enc_x, b_enc_x):
    N = x.shape[0]
    H = W_enc_x.shape[1]

    def body(x_r, wx_r, bx_r, n_o):
        n_o[...] = jnp.maximum(_dot(x_r[...], wx_r[...]) + bx_r[...], 0.0)

    return pl.pallas_call(
        body, out_shape=jax.ShapeDtypeStruct((N, H), F32),
    )(x, W_enc_x, b_enc_x)


def _edge_mlp(ns, nd, edges, Wsd, We, b, enc=None):
    E = ns.shape[0]
    H = ns.shape[1]

    if enc is None:
        def body(ns_r, nd_r, e_r, wsd_r, we_r, b_r, o_r):
            e = e_r[...]
            nsnd = jnp.concatenate([ns_r[...], nd_r[...]], axis=1)
            pre = (_dot(nsnd, wsd_r[...]) + _dot(e, we_r[...])) + b_r[...]
            o_r[...] = e + jnp.maximum(pre, 0.0)

        return pl.pallas_call(
            body, grid=(E // EB,),
            in_specs=[pl.BlockSpec((EB, H), lambda i: (i, 0)),
                      pl.BlockSpec((EB, H), lambda i: (i, 0)),
                      pl.BlockSpec((EB, H), lambda i: (i, 0)),
                      pl.BlockSpec((2 * H, H), lambda i: (0, 0)),
                      pl.BlockSpec((H, H), lambda i: (0, 0)),
                      pl.BlockSpec((1, H), lambda i: (0, 0))],
            out_specs=pl.BlockSpec((EB, H), lambda i: (i, 0)),
            out_shape=jax.ShapeDtypeStruct((E, H), F32),
        )(ns, nd, edges, Wsd, We, b)

    edge_attr, W_enc_e, b_enc_e = enc
    DE = edge_attr.shape[1]

    def body0(ns_r, nd_r, ea_r, wenc_r, benc_r, wsd_r, we_r, b_r, o_r):
        e = jnp.maximum(_dot(ea_r[...], wenc_r[...]) + benc_r[...], 0.0)
        nsnd = jnp.concatenate([ns_r[...], nd_r[...]], axis=1)
        pre = (_dot(nsnd, wsd_r[...]) + _dot(e, we_r[...])) + b_r[...]
        o_r[...] = e + jnp.maximum(pre, 0.0)

    return pl.pallas_call(
        body0, grid=(E // EB,),
        in_specs=[pl.BlockSpec((EB, H), lambda i: (i, 0)),
                  pl.BlockSpec((EB, H), lambda i: (i, 0)),
                  pl.BlockSpec((EB, DE), lambda i: (i, 0)),
                  pl.BlockSpec((DE, H), lambda i: (0, 0)),
                  pl.BlockSpec((1, H), lambda i: (0, 0)),
                  pl.BlockSpec((2 * H, H), lambda i: (0, 0)),
                  pl.BlockSpec((H, H), lambda i: (0, 0)),
                  pl.BlockSpec((1, H), lambda i: (0, 0))],
        out_specs=pl.BlockSpec((EB, H), lambda i: (i, 0)),
        out_shape=jax.ShapeDtypeStruct((E, H), F32),
    )(ns, nd, edge_attr, W_enc_e, b_enc_e, Wsd, We, b)


def _node_mlp(nodes, agg2, Wnode, bn):
    N, H = nodes.shape
    NP2 = agg2.shape[0]
    G = 5
    RB = N // G

    def body(n_r, a_r, w_r, b_r, o_r):
        n = n_r[...]
        n_in = jnp.concatenate([n, a_r[...]], axis=1)
        o_r[...] = n + jnp.maximum(_dot(n_in, w_r[...]) + b_r[...], 0.0)

    return pl.pallas_call(
        body, grid=(G,),
        in_specs=[pl.BlockSpec((RB, H), lambda i: (i, 0)),
                  pl.BlockSpec((RB, H), lambda i: (i, 0)),
                  pl.BlockSpec((2 * H, H), lambda i: (0, 0)),
                  pl.BlockSpec((1, H), lambda i: (0, 0))],
        out_specs=pl.BlockSpec((RB, H), lambda i: (i, 0)),
        out_shape=jax.ShapeDtypeStruct((N, H), F32),
    )(nodes, agg2, Wnode, bn)


def _node_decode(nodes, agg2, Wnode, bn, W_dec_pad, b_dec_pad, bus_shunt,
                 slack_idx):
    N, H = nodes.shape

    G = 5
    RB = N // G

    def body(n_r, a_r, w_r, bnr, wdec_r, bdec_r, sh_r, sl_r,
             bus_o, volt_o, si_o):
        i = pl.program_id(0)
        n = n_r[...]
        n_in = jnp.concatenate([n, a_r[...]], axis=1)
        nn = n + jnp.maximum(_dot(n_in, w_r[...]) + bnr[...], 0.0)
        busw = _dot(nn, wdec_r[...]) + bdec_r[...]
        bus_o[...] = busw
        va = busw[:, 0:1]
        vm = busw[:, 1:2]
        vre = vm * jnp.cos(va)
        vim = vm * jnp.sin(va)
        colk2 = lax.broadcasted_iota(jnp.int32, (RB, 128), 1)
        volt_o[...] = (jnp.where(colk2 == 0, vre, 0.0)
                       + jnp.where(colk2 == 1, vim, 0.0))
        rows = lax.broadcasted_iota(jnp.int32, (RB, 1), 0) + i * RB
        msk = rows == sl_r[0]
        vm2 = vm * vm
        p = jnp.sum(jnp.where(msk, vm2 * sh_r[:, 1:2], 0.0))
        q = jnp.sum(jnp.where(msk, vm2 * sh_r[:, 0:1], 0.0))
        colk = lax.broadcasted_iota(jnp.int32, (8, 128), 1)
        contrib = (jnp.where(colk == 0, p, 0.0)
                   + jnp.where(colk == 1, q, 0.0))

        @pl.when(i == 0)
        def _():
            si_o[...] = contrib

        @pl.when(i != 0)
        def _():
            si_o[...] = si_o[...] + contrib

    return pl.pallas_call(
        body, grid=(G,),
        in_specs=[
            pl.BlockSpec((RB, H), lambda i: (i, 0)),
            pl.BlockSpec((RB, H), lambda i: (i, 0)),
            pl.BlockSpec((2 * H, H), lambda i: (0, 0)),
            pl.BlockSpec((1, H), lambda i: (0, 0)),
            pl.BlockSpec((H, 128), lambda i: (0, 0)),
            pl.BlockSpec((1, 128), lambda i: (0, 0)),
            pl.BlockSpec((RB, 2), lambda i: (i, 0)),
            pl.BlockSpec(memory_space=pltpu.SMEM),
        ],
        out_specs=[pl.BlockSpec((RB, 128), lambda i: (i, 0)),
                   pl.BlockSpec((RB, 128), lambda i: (i, 0)),
                   pl.BlockSpec((8, 128), lambda i: (0, 0))],
        out_shape=[jax.ShapeDtypeStruct((N, 128), F32),
                   jax.ShapeDtypeStruct((N, 128), F32),
                   jax.ShapeDtypeStruct((8, 128), F32)],
    )(nodes, agg2, Wnode, bn, W_dec_pad, b_dec_pad, bus_shunt, slack_idx)


def _flows(edge_attr, vriF, viiF, vrjF, vijF, srcF, dstF, slack_idx,
           slack_init):
    E, DE = edge_attr.shape

    def body(ea_r, vri_r, vii_r, vrj_r, vij_r, s_r, d_r, sl_r, si_r,
             ep_o, sa_o):
        ea = ea_r[...]
        r = ea[:, 0]
        xx = ea[:, 1]
        bfr = ea[:, 3]
        bto = ea[:, 5]
        tap = ea[:, 6]
        shift = ea[:, 7]
        vri = vri_r[0, 0, :]
        vii = vii_r[0, 0, :]
        vrj = vrj_r[0, 0, :]
        vij = vij_r[0, 0, :]
        den = r * r + xx * xx
        gy = r / den
        by = -xx / den
        cs = jnp.cos(shift)
        sn = jnp.sin(shift)
        ai = vri * vri + vii * vii
        aj = vrj * vrj + vij * vij
        mr = vri * vrj + vii * vij
        mi = vii * vrj - vri * vij
        itap = 1.0 / tap
        itap2 = itap * itap
        p1 = gy * mr + by * mi
        q1 = gy * mi - by * mr
        sfr_re = gy * ai * itap2 - (p1 * cs + q1 * sn) * itap
        sfr_im = -(by + bfr) * ai * itap2 - (q1 * cs - p1 * sn) * itap
        p2 = gy * mr - by * mi
        q2 = -(gy * mi + by * mr)
        sto_re = gy * aj - (p2 * cs - q2 * sn) * itap
        sto_im = -(by + bto) * aj - (p2 * sn + q2 * cs) * itap
        ep_o[...] = jnp.concatenate([sfr_re[:, None], sfr_im[:, None],
                                     sto_re[:, None], sto_im[:, None]], axis=1)
        sl = sl_r[0]
        ms = s_r[0, 0, :] == sl
        md = d_r[0, 0, :] == sl
        p = (jnp.sum(jnp.where(ms, sfr_re, 0.0))
             + jnp.sum(jnp.where(md, sto_re, 0.0)))
        q = (jnp.sum(jnp.where(ms, sfr_im, 0.0))
             + jnp.sum(jnp.where(md, sto_im, 0.0)))
        colk = lax.broadcasted_iota(jnp.int32, (8, 128), 1)
        contrib = (jnp.where(colk == 0, p, 0.0)
                   + jnp.where(colk == 1, q, 0.0))
        i = pl.program_id(0)

        @pl.when(i == 0)
        def _():
            sa_o[...] = si_r[...] + contrib

        @pl.when(i != 0)
        def _():
            sa_o[...] = sa_o[...] + contrib

    return pl.pallas_call(
        body, grid=(E // EB,),
        in_specs=[pl.BlockSpec((EB, DE), lambda i: (i, 0))]
                 + [pl.BlockSpec((1, 1, EB), lambda i: (i, 0, 0))] * 6
                 + [pl.BlockSpec(memory_space=pltpu.SMEM),
                    pl.BlockSpec((8, 128), lambda i: (0, 0))],
        out_specs=[pl.BlockSpec((EB, 4), lambda i: (i, 0)),
                   pl.BlockSpec((8, 128), lambda i: (0, 0))],
        out_shape=[jax.ShapeDtypeStruct((E, 4), F32),
                   jax.ShapeDtypeStruct((8, 128), F32)],
    )(edge_attr, vriF, viiF, vrjF, vijF, srcF, dstF, slack_idx, slack_init)


def kernel(x, edge_attr, bus_shunt, W_enc_x, b_enc_x, W_enc_e, b_enc_e,
           W_edge, b_edge, W_node, b_node, W_dec, b_dec, edge_index,
           slack_idx):
    N = x.shape[0]
    H = W_enc_x.shape[1]
    E = edge_attr.shape[0]
    K = W_edge.shape[0]

    src = edge_index[0]
    dst = edge_index[1]
    n_chunks = (E // NW) // CH
    src3 = src.reshape(NW, n_chunks, CH)
    dst3 = dst.reshape(NW, n_chunks, CH)
    srcF = src.reshape(E // EB, 1, EB)
    dstF = dst.reshape(E // EB, 1, EB)
    srcW = src.reshape(NW, E // NW)
    dstW = dst.reshape(NW, E // NW)

    perm = jnp.argsort(dst, stable=True).astype(jnp.int32)
    dsort = dst[perm]
    sbounds = jnp.searchsorted(dsort, jnp.arange(0, NW * RPT + 1, RPT,
                                                 dtype=jnp.int32)
                               ).astype(jnp.int32)
    sbounds = jnp.concatenate([sbounds, jnp.full((15,), E, jnp.int32)])
    perm_pad = jnp.concatenate([perm, jnp.zeros((128,), jnp.int32)])
    dsort_pad = jnp.concatenate([dsort, jnp.zeros((128,), jnp.int32)])

    be = [b_edge[l].reshape(1, H) for l in range(K)]
    bn = [b_node[l].reshape(1, H) for l in range(K)]

    nodes = _encode_nodes(x, W_enc_x, b_enc_x.reshape(1, H))
    edges = None
    for l in range(K):
        ns, nd = _gather_pair(nodes, src3, dst3)
        if l == 0:
            edges = _edge_mlp(ns, nd, None, W_edge[l, :2 * H],
                              W_edge[l, 2 * H:], be[l],
                              enc=(edge_attr, W_enc_e, b_enc_e.reshape(1, H)))
        else:
            edges = _edge_mlp(ns, nd, edges, W_edge[l, :2 * H],
                              W_edge[l, 2 * H:], be[l])
        aggf = _scatter_sorted(edges, perm_pad, dsort_pad, sbounds)
        agg = aggf.reshape(NW * RPT, H)[:N]
        if l < K - 1:
            nodes = _node_mlp(nodes, agg, W_node[l], bn[l])
        else:
            busw, voltw, slack_init = _node_decode(
                nodes, agg, W_node[l], bn[l],
                jnp.pad(W_dec, ((0, 0), (0, 126))),
                jnp.pad(b_dec.reshape(1, 2), ((0, 0), (0, 126))),
                bus_shunt, slack_idx)

    bus = busw[:, 0:2]
    vt = voltw[:, 0:2].T
    vri, vii, vrj, vij = _gather_volt(vt, srcW, dstW)
    nF = E // EB
    edge_preds, slack_acc = _flows(edge_attr,
                                   vri.reshape(nF, 1, EB),
                                   vii.reshape(nF, 1, EB),
                                   vrj.reshape(nF, 1, EB),
                                   vij.reshape(nF, 1, EB),
                                   srcF, dstF, slack_idx, slack_init)
    slack = slack_acc[0:1, 0:2]
    return (bus, edge_preds, slack)
